# baseline (speedup 1.0000x reference)
import functools

import jax
import jax.numpy as jnp
from jax.experimental import pallas as pl
from jax.experimental.pallas import tpu as pltpu

B, S, D = 2, 2048, 768
HEADS = 4
K = 8
SEM = 512
N = B * S

TI = 256
NI = S // TI
TR = 512

_f32 = jnp.float32


def _dot(a, b, ca, cb):
    return jax.lax.dot_general(
        a, b, (((ca,), (cb,)), ((), ())), preferred_element_type=_f32)


def _leaky(x):
    return jnp.where(x >= 0, x, 0.2 * x)


def _proj_kernel(x_ref, pw_ref, sw_ref, ph_ref, sh_ref):
    x = x_ref[...]
    ph_ref[...] = _dot(x, pw_ref[...], 1, 1)
    sh_ref[...] = _dot(x, sw_ref[...], 1, 1)


def _run_proj(x, phi_W, psi_W):
    return pl.pallas_call(
        _proj_kernel,
        grid=(N // TR,),
        in_specs=[
            pl.BlockSpec((TR, D), lambda i: (i, 0)),
            pl.BlockSpec((D, D), lambda i: (0, 0)),
            pl.BlockSpec((D, D), lambda i: (0, 0)),
        ],
        out_specs=[
            pl.BlockSpec((TR, D), lambda i: (i, 0)),
            pl.BlockSpec((TR, D), lambda i: (i, 0)),
        ],
        out_shape=[
            jax.ShapeDtypeStruct((N, D), _f32),
            jax.ShapeDtypeStruct((N, D), _f32),
        ],
    )(x, phi_W, psi_W)


def _topk_kernel(ph_ref, sh_ref, idx_ref, pm_ref):
    i = pl.program_id(1)
    phi = ph_ref[0]
    psi = sh_ref[0]
    aff = jnp.exp(_dot(phi, psi, 1, 1))
    iota = jax.lax.broadcasted_iota(jnp.int32, (TI, S), 1)
    work = aff
    vals = []
    idxs = []
    for _ in range(K):
        m = jnp.max(work, axis=1, keepdims=True)
        amin = jnp.min(jnp.where(work == m, iota, S), axis=1,
                       keepdims=True)
        vals.append(m)
        idxs.append(amin)
        work = jnp.where(iota == amin, -1.0, work)
    v = jnp.concatenate(vals, axis=1)
    ix = jnp.concatenate(idxs, axis=1)
    w = v / (jnp.sum(v, axis=1, keepdims=True) + 1e-8)
    row = i * TI + jax.lax.broadcasted_iota(jnp.int32, (TI, K), 0)
    mask = (ix != row).astype(_f32)
    pm_ref[0] = jnp.clip(w, 1e-8, None) * mask
    idx_ref[0] = ix


def _run_topk(phi_h, psi_h):
    ph = phi_h.reshape(B, S, D)
    sh = psi_h.reshape(B, S, D)
    return pl.pallas_call(
        _topk_kernel,
        grid=(B, NI),
        in_specs=[
            pl.BlockSpec((1, TI, D), lambda b, i: (b, i, 0)),
            pl.BlockSpec((1, S, D), lambda b, i: (b, 0, 0)),
        ],
        out_specs=[
            pl.BlockSpec((1, TI, K), lambda b, i: (b, i, 0)),
            pl.BlockSpec((1, TI, K), lambda b, i: (b, i, 0)),
        ],
        out_shape=[
            jax.ShapeDtypeStruct((B, S, K), jnp.int32),
            jax.ShapeDtypeStruct((B, S, K), _f32),
        ],
    )(ph, sh)


def _gat_proj_kernel(x_ref, w_ref, as_ref, ad_ref, h_ref, ab_ref):
    hflat = _dot(x_ref[...], w_ref[...], 1, 1)
    h3 = hflat.reshape(TR, HEADS, D)
    a = jnp.sum(h3 * as_ref[...][None], axis=2)
    b = jnp.sum(h3 * ad_ref[...][None], axis=2)
    for hh in range(HEADS):
        h_ref[hh] = h3[:, hh, :].astype(jnp.bfloat16)
    ab_ref[...] = jnp.concatenate([a, b], axis=1)


def _run_gat_proj(x, W, att_s, att_d):
    return pl.pallas_call(
        _gat_proj_kernel,
        grid=(N // TR,),
        in_specs=[
            pl.BlockSpec((TR, D), lambda i: (i, 0)),
            pl.BlockSpec((HEADS * D, D), lambda i: (0, 0)),
            pl.BlockSpec((HEADS, D), lambda i: (0, 0)),
            pl.BlockSpec((HEADS, D), lambda i: (0, 0)),
        ],
        out_specs=[
            pl.BlockSpec((HEADS, TR, D), lambda i: (0, i, 0)),
            pl.BlockSpec((TR, 2 * HEADS), lambda i: (i, 0)),
        ],
        out_shape=[
            jax.ShapeDtypeStruct((HEADS, N, D), jnp.bfloat16),
            jax.ShapeDtypeStruct((N, 2 * HEADS), _f32),
        ],
    )(x, W, att_s, att_d)


_bf16 = jnp.bfloat16


def _edgew_kernel(ab_ref, abf_ref, idx_ref, pm_ref, w_ref):
    a = ab_ref[...][:, :HEADS]
    bfull = abf_ref[...][:, HEADS:]
    idx = idx_ref[0]
    pm = pm_ref[0]
    iota = jax.lax.broadcasted_iota(jnp.int32, (TI, S), 1)
    cols = []
    for k in range(K):
        mk = (iota == idx[:, k][:, None]).astype(_f32)
        bg = _dot(mk, bfull, 1, 0)
        cols.append(pm[:, k][:, None] * jnp.exp(_leaky(a + bg)))
    w_ref[0] = jnp.concatenate(cols, axis=1)


def _run_edgew(ab, idx, pm):
    return pl.pallas_call(
        _edgew_kernel,
        grid=(B, NI),
        in_specs=[
            pl.BlockSpec((TI, 2 * HEADS), lambda b, i: (b * NI + i, 0)),
            pl.BlockSpec((S, 2 * HEADS), lambda b, i: (b, 0)),
            pl.BlockSpec((1, TI, K), lambda b, i: (b, i, 0)),
            pl.BlockSpec((1, TI, K), lambda b, i: (b, i, 0)),
        ],
        out_specs=pl.BlockSpec((1, TI, K * HEADS), lambda b, i: (b, i, 0)),
        out_shape=jax.ShapeDtypeStruct((B, S, K * HEADS), _f32),
    )(ab, ab, idx, pm)


TJ = 256
NJ = S // TJ


def _agg_kernel(h_ref, w_ref, idx_ref, out_ref, attn_ref):
    j = pl.program_id(1)

    wt = w_ref[0]
    idx16 = idx_ref[0].astype(jnp.int16)

    base = (j * TJ).astype(jnp.int16)
    iota = base + jax.lax.broadcasted_iota(jnp.int16, (S, TJ), 1)
    E = [jnp.zeros((S, TJ), _bf16) for _ in range(HEADS)]
    for k in range(K):
        match = iota == idx16[:, k][:, None]
        for hh in range(HEADS):
            wk = wt[:, k * HEADS + hh][:, None].astype(_bf16)
            E[hh] = jnp.where(match, wk, E[hh])

    ones = jnp.ones((S, 128), _bf16)
    for hh in range(HEADS):
        x = jnp.concatenate([h_ref[hh], ones], axis=1)
        acc = _dot(E[hh], x, 0, 0)
        out_ref[0, :, hh, :] = acc[:, :D]
        attn_ref[0, :, hh, :] = acc[:, D:D + 1]


def _run_agg(h, w, idx):
    return pl.pallas_call(
        _agg_kernel,
        grid=(B, NJ),
        in_specs=[
            pl.BlockSpec((HEADS, S, D), lambda b, j: (0, b, 0)),
            pl.BlockSpec((1, S, K * HEADS), lambda b, j: (b, 0, 0)),
            pl.BlockSpec((1, S, K), lambda b, j: (b, 0, 0)),
        ],
        out_specs=[
            pl.BlockSpec((1, TJ, HEADS, D), lambda b, j: (b, j, 0, 0)),
            pl.BlockSpec((1, TJ, HEADS, 1), lambda b, j: (b, j, 0, 0)),
        ],
        out_shape=[
            jax.ShapeDtypeStruct((B, S, HEADS, D), _f32),
            jax.ShapeDtypeStruct((B, S, HEADS, 1), _f32),
        ],
    )(h, w, idx)


TS = 512


def _norm_kernel(out_ref, attn_ref, x_ref):
    o = out_ref[0]
    at = attn_ref[0, :, :, 0]
    xn = jnp.mean(o / (at[:, :, None] + 1e-8), axis=1)
    x_ref[0] = jnp.maximum(xn, 0.0)


def _run_norm(out, attn):
    return pl.pallas_call(
        _norm_kernel,
        grid=(B, S // TS),
        in_specs=[
            pl.BlockSpec((1, TS, HEADS, D), lambda b, j: (b, j, 0, 0)),
            pl.BlockSpec((1, TS, HEADS, 1), lambda b, j: (b, j, 0, 0)),
        ],
        out_specs=pl.BlockSpec((1, TS, D), lambda b, j: (b, j, 0)),
        out_shape=jax.ShapeDtypeStruct((B, S, D), _f32),
    )(out, attn)


def _head_kernel(x_ref, pw_ref, w1_ref, b1_ref, w2_ref, b2_ref, o_ref):
    pooled = []
    for b in range(B):
        xb = x_ref[b]
        p = _dot(xb, pw_ref[...], 1, 1)
        p = p - jnp.max(p, axis=0, keepdims=True)
        al = jnp.exp(p)
        al = al / jnp.sum(al, axis=0, keepdims=True)
        pooled.append(_dot(al, xb, 0, 0))
    pooled = jnp.concatenate(pooled, axis=0)
    hmid = jnp.maximum(_dot(pooled, w1_ref[...], 1, 1) + b1_ref[...], 0.0)
    o_ref[...] = _dot(hmid, w2_ref[...], 1, 1) + b2_ref[...]


def _run_head(x, pool_W, W1, b1, W2, b2):
    return pl.pallas_call(
        _head_kernel,
        grid=(1,),
        in_specs=[
            pl.BlockSpec((B, S, D), lambda i: (0, 0, 0)),
            pl.BlockSpec((1, D), lambda i: (0, 0)),
            pl.BlockSpec((D // 2, D), lambda i: (0, 0)),
            pl.BlockSpec((1, D // 2), lambda i: (0, 0)),
            pl.BlockSpec((SEM, D // 2), lambda i: (0, 0)),
            pl.BlockSpec((1, SEM), lambda i: (0, 0)),
        ],
        out_specs=pl.BlockSpec((B, SEM), lambda i: (0, 0)),
        out_shape=jax.ShapeDtypeStruct((B, SEM), _f32),
    )(x, pool_W, W1, b1.reshape(1, -1), W2, b2.reshape(1, -1))


@jax.jit
def kernel(hidden_states, phi_W, psi_W, gat1_W, gat1_att, gat2_W, gat2_att,
           pool_W, proj_W1, proj_b1, proj_W2, proj_b2):
    x = hidden_states.reshape(N, D)
    phi_h, psi_h = _run_proj(x, phi_W, psi_W)
    idx, pm = _run_topk(phi_h, psi_h)

    for (W, att) in ((gat1_W, gat1_att), (gat2_W, gat2_att)):
        att_s = att[0, :, :D]
        att_d = att[0, :, D:]
        h, ab = _run_gat_proj(x, W, att_s, att_d)
        w = _run_edgew(ab, idx, pm)
        out, attn = _run_agg(h, w, idx)
        xb = _run_norm(out, attn)
        x = xb.reshape(N, D)

    return _run_head(xb, pool_W, proj_W1, proj_b1, proj_W2, proj_b2)


# speedup vs baseline: 1.2200x; 1.2200x over previous
import functools

import jax
import jax.numpy as jnp
from jax.experimental import pallas as pl
from jax.experimental.pallas import tpu as pltpu

B, S, D = 2, 2048, 768
HEADS = 4
K = 8
SEM = 512
N = B * S

TI = 256
NI = S // TI
TR = 512

_f32 = jnp.float32


def _dot(a, b, ca, cb):
    return jax.lax.dot_general(
        a, b, (((ca,), (cb,)), ((), ())), preferred_element_type=_f32)


def _leaky(x):
    return jnp.where(x >= 0, x, 0.2 * x)


def _proj_kernel(x_ref, pw_ref, sw_ref, ph_ref, sh_ref):
    x = x_ref[...]
    ph_ref[...] = _dot(x, pw_ref[...], 1, 1)
    sh_ref[...] = _dot(x, sw_ref[...], 1, 1)


def _run_proj(x, phi_W, psi_W):
    return pl.pallas_call(
        _proj_kernel,
        grid=(N // TR,),
        in_specs=[
            pl.BlockSpec((TR, D), lambda i: (i, 0)),
            pl.BlockSpec((D, D), lambda i: (0, 0)),
            pl.BlockSpec((D, D), lambda i: (0, 0)),
        ],
        out_specs=[
            pl.BlockSpec((TR, D), lambda i: (i, 0)),
            pl.BlockSpec((TR, D), lambda i: (i, 0)),
        ],
        out_shape=[
            jax.ShapeDtypeStruct((N, D), _f32),
            jax.ShapeDtypeStruct((N, D), _f32),
        ],
    )(x, phi_W, psi_W)


def _topk_kernel(ph_ref, sh_ref, idx_ref, pm_ref):
    i = pl.program_id(1)
    phi = ph_ref[0]
    psi = sh_ref[0]
    aff = jnp.exp(_dot(phi, psi, 1, 1))
    iota = jax.lax.broadcasted_iota(jnp.int32, (TI, S), 1)
    work = aff
    vals = []
    idxs = []
    for _ in range(K):
        m = jnp.max(work, axis=1, keepdims=True)
        amin = jnp.min(jnp.where(work == m, iota, S), axis=1,
                       keepdims=True)
        vals.append(m)
        idxs.append(amin)
        work = jnp.where(iota == amin, -1.0, work)
    v = jnp.concatenate(vals, axis=1)
    ix = jnp.concatenate(idxs, axis=1)
    w = v / (jnp.sum(v, axis=1, keepdims=True) + 1e-8)
    row = i * TI + jax.lax.broadcasted_iota(jnp.int32, (TI, K), 0)
    mask = (ix != row).astype(_f32)
    pm_ref[0] = jnp.clip(w, 1e-8, None) * mask
    idx_ref[0] = ix


def _run_topk(phi_h, psi_h):
    ph = phi_h.reshape(B, S, D)
    sh = psi_h.reshape(B, S, D)
    return pl.pallas_call(
        _topk_kernel,
        grid=(B, NI),
        in_specs=[
            pl.BlockSpec((1, TI, D), lambda b, i: (b, i, 0)),
            pl.BlockSpec((1, S, D), lambda b, i: (b, 0, 0)),
        ],
        out_specs=[
            pl.BlockSpec((1, TI, K), lambda b, i: (b, i, 0)),
            pl.BlockSpec((1, TI, K), lambda b, i: (b, i, 0)),
        ],
        out_shape=[
            jax.ShapeDtypeStruct((B, S, K), jnp.int32),
            jax.ShapeDtypeStruct((B, S, K), _f32),
        ],
    )(ph, sh)


def _gat_proj_kernel(x_ref, w_ref, as_ref, ad_ref, h_ref, ab_ref):
    hflat = _dot(x_ref[...], w_ref[...], 1, 1)
    h3 = hflat.reshape(TR, HEADS, D)
    a = jnp.sum(h3 * as_ref[...][None], axis=2)
    b = jnp.sum(h3 * ad_ref[...][None], axis=2)
    for hh in range(HEADS):
        h_ref[hh] = h3[:, hh, :].astype(jnp.bfloat16)
    ab_ref[...] = jnp.concatenate([a, b], axis=1)


def _run_gat_proj(x, W, att_s, att_d):
    return pl.pallas_call(
        _gat_proj_kernel,
        grid=(N // TR,),
        in_specs=[
            pl.BlockSpec((TR, D), lambda i: (i, 0)),
            pl.BlockSpec((HEADS * D, D), lambda i: (0, 0)),
            pl.BlockSpec((HEADS, D), lambda i: (0, 0)),
            pl.BlockSpec((HEADS, D), lambda i: (0, 0)),
        ],
        out_specs=[
            pl.BlockSpec((HEADS, TR, D), lambda i: (0, i, 0)),
            pl.BlockSpec((TR, 2 * HEADS), lambda i: (i, 0)),
        ],
        out_shape=[
            jax.ShapeDtypeStruct((HEADS, N, D), jnp.bfloat16),
            jax.ShapeDtypeStruct((N, 2 * HEADS), _f32),
        ],
    )(x, W, att_s, att_d)


_bf16 = jnp.bfloat16


def _edgew_kernel(ab_ref, abf_ref, idx_ref, pm_ref, w_ref):
    a = ab_ref[...][:, :HEADS]
    bfull = abf_ref[...][:, HEADS:]
    idx = idx_ref[0]
    pm = pm_ref[0]
    iota = jax.lax.broadcasted_iota(jnp.int32, (TI, S), 1)
    cols = []
    for k in range(K):
        mk = (iota == idx[:, k][:, None]).astype(_f32)
        bg = _dot(mk, bfull, 1, 0)
        cols.append(pm[:, k][:, None] * jnp.exp(_leaky(a + bg)))
    w_ref[0] = jnp.concatenate(cols, axis=1)


def _run_edgew(ab, idx, pm):
    return pl.pallas_call(
        _edgew_kernel,
        grid=(B, NI),
        in_specs=[
            pl.BlockSpec((TI, 2 * HEADS), lambda b, i: (b * NI + i, 0)),
            pl.BlockSpec((S, 2 * HEADS), lambda b, i: (b, 0)),
            pl.BlockSpec((1, TI, K), lambda b, i: (b, i, 0)),
            pl.BlockSpec((1, TI, K), lambda b, i: (b, i, 0)),
        ],
        out_specs=pl.BlockSpec((1, TI, K * HEADS), lambda b, i: (b, i, 0)),
        out_shape=jax.ShapeDtypeStruct((B, S, K * HEADS), _f32),
    )(ab, ab, idx, pm)


def _agg_kernel(h_ref, w_ref, idx_ref, out_ref, attn_ref):
    hid = pl.program_id(1)

    htile = h_ref[0]
    wt = w_ref[0]
    idx16 = idx_ref[0].astype(jnp.int16)

    iota = jax.lax.broadcasted_iota(jnp.int16, (S, S), 1)
    cid = jax.lax.broadcasted_iota(jnp.int32, (1, K * HEADS), 1)
    E = jnp.zeros((S, S), _bf16)
    for k in range(K):
        match = iota == idx16[:, k][:, None]
        sel = (cid == k * HEADS + hid).astype(_f32)
        wk = jnp.sum(wt * sel, axis=1, keepdims=True)
        E = jnp.where(match, wk.astype(_bf16), E)

    x = jnp.concatenate([htile, jnp.ones((S, 128), _bf16)], axis=1)
    acc = _dot(E, x, 0, 0)
    out_ref[0, 0] = acc[:, :D]
    attn_ref[0, 0] = acc[:, D:D + 1]


def _run_agg(h, w, idx):
    return pl.pallas_call(
        _agg_kernel,
        grid=(B, HEADS),
        in_specs=[
            pl.BlockSpec((1, S, D), lambda b, h: (h, b, 0)),
            pl.BlockSpec((1, S, K * HEADS), lambda b, h: (b, 0, 0)),
            pl.BlockSpec((1, S, K), lambda b, h: (b, 0, 0)),
        ],
        out_specs=[
            pl.BlockSpec((1, 1, S, D), lambda b, h: (b, h, 0, 0)),
            pl.BlockSpec((1, 1, S, 1), lambda b, h: (b, h, 0, 0)),
        ],
        out_shape=[
            jax.ShapeDtypeStruct((B, HEADS, S, D), _f32),
            jax.ShapeDtypeStruct((B, HEADS, S, 1), _f32),
        ],
    )(h, w, idx)


TS = 512


def _norm_kernel(out_ref, attn_ref, x_ref):
    o = out_ref[0]
    at = attn_ref[0, :, :, 0]
    xn = jnp.mean(o / (at[:, :, None] + 1e-8), axis=0)
    x_ref[0] = jnp.maximum(xn, 0.0)


def _run_norm(out, attn):
    return pl.pallas_call(
        _norm_kernel,
        grid=(B, S // TS),
        in_specs=[
            pl.BlockSpec((1, HEADS, TS, D), lambda b, j: (b, 0, j, 0)),
            pl.BlockSpec((1, HEADS, TS, 1), lambda b, j: (b, 0, j, 0)),
        ],
        out_specs=pl.BlockSpec((1, TS, D), lambda b, j: (b, j, 0)),
        out_shape=jax.ShapeDtypeStruct((B, S, D), _f32),
    )(out, attn)


def _head_kernel(x_ref, pw_ref, w1_ref, b1_ref, w2_ref, b2_ref, o_ref):
    pooled = []
    for b in range(B):
        xb = x_ref[b]
        p = _dot(xb, pw_ref[...], 1, 1)
        p = p - jnp.max(p, axis=0, keepdims=True)
        al = jnp.exp(p)
        al = al / jnp.sum(al, axis=0, keepdims=True)
        pooled.append(_dot(al, xb, 0, 0))
    pooled = jnp.concatenate(pooled, axis=0)
    hmid = jnp.maximum(_dot(pooled, w1_ref[...], 1, 1) + b1_ref[...], 0.0)
    o_ref[...] = _dot(hmid, w2_ref[...], 1, 1) + b2_ref[...]


def _run_head(x, pool_W, W1, b1, W2, b2):
    return pl.pallas_call(
        _head_kernel,
        grid=(1,),
        in_specs=[
            pl.BlockSpec((B, S, D), lambda i: (0, 0, 0)),
            pl.BlockSpec((1, D), lambda i: (0, 0)),
            pl.BlockSpec((D // 2, D), lambda i: (0, 0)),
            pl.BlockSpec((1, D // 2), lambda i: (0, 0)),
            pl.BlockSpec((SEM, D // 2), lambda i: (0, 0)),
            pl.BlockSpec((1, SEM), lambda i: (0, 0)),
        ],
        out_specs=pl.BlockSpec((B, SEM), lambda i: (0, 0)),
        out_shape=jax.ShapeDtypeStruct((B, SEM), _f32),
    )(x, pool_W, W1, b1.reshape(1, -1), W2, b2.reshape(1, -1))


@jax.jit
def kernel(hidden_states, phi_W, psi_W, gat1_W, gat1_att, gat2_W, gat2_att,
           pool_W, proj_W1, proj_b1, proj_W2, proj_b2):
    x = hidden_states.reshape(N, D)
    phi_h, psi_h = _run_proj(x, phi_W, psi_W)
    idx, pm = _run_topk(phi_h, psi_h)

    for (W, att) in ((gat1_W, gat1_att), (gat2_W, gat2_att)):
        att_s = att[0, :, :D]
        att_d = att[0, :, D:]
        h, ab = _run_gat_proj(x, W, att_s, att_d)
        w = _run_edgew(ab, idx, pm)
        out, attn = _run_agg(h, w, idx)
        xb = _run_norm(out, attn)
        x = xb.reshape(N, D)

    return _run_head(xb, pool_W, proj_W1, proj_b1, proj_W2, proj_b2)


# speedup vs baseline: 1.3301x; 1.0903x over previous
import functools

import jax
import jax.numpy as jnp
from jax.experimental import pallas as pl
from jax.experimental.pallas import tpu as pltpu

B, S, D = 2, 2048, 768
HEADS = 4
K = 8
SEM = 512
N = B * S

TI = 256
NI = S // TI
TR = 512

_f32 = jnp.float32


def _dot(a, b, ca, cb):
    return jax.lax.dot_general(
        a, b, (((ca,), (cb,)), ((), ())), preferred_element_type=_f32)


def _leaky(x):
    return jnp.where(x >= 0, x, 0.2 * x)


def _proj_kernel(x_ref, pw_ref, sw_ref, ph_ref, sh_ref):
    x = x_ref[...]
    ph_ref[...] = _dot(x, pw_ref[...], 1, 1)
    sh_ref[...] = _dot(x, sw_ref[...], 1, 1)


def _run_proj(x, phi_W, psi_W):
    return pl.pallas_call(
        _proj_kernel,
        grid=(N // TR,),
        in_specs=[
            pl.BlockSpec((TR, D), lambda i: (i, 0)),
            pl.BlockSpec((D, D), lambda i: (0, 0)),
            pl.BlockSpec((D, D), lambda i: (0, 0)),
        ],
        out_specs=[
            pl.BlockSpec((TR, D), lambda i: (i, 0)),
            pl.BlockSpec((TR, D), lambda i: (i, 0)),
        ],
        out_shape=[
            jax.ShapeDtypeStruct((N, D), _f32),
            jax.ShapeDtypeStruct((N, D), _f32),
        ],
    )(x, phi_W, psi_W)


def _topk_kernel(ph_ref, sh_ref, idx_ref, pm_ref):
    i = pl.program_id(1)
    phi = ph_ref[0]
    psi = sh_ref[0]
    aff = jnp.exp(_dot(phi, psi, 1, 1))
    iota = jax.lax.broadcasted_iota(jnp.int32, (TI, S), 1)
    work = aff
    vals = []
    idxs = []
    for _ in range(K):
        m = jnp.max(work, axis=1, keepdims=True)
        amin = jnp.min(jnp.where(work == m, iota, S), axis=1,
                       keepdims=True)
        vals.append(m)
        idxs.append(amin)
        work = jnp.where(iota == amin, -1.0, work)
    v = jnp.concatenate(vals, axis=1)
    ix = jnp.concatenate(idxs, axis=1)
    w = v / (jnp.sum(v, axis=1, keepdims=True) + 1e-8)
    row = i * TI + jax.lax.broadcasted_iota(jnp.int32, (TI, K), 0)
    mask = (ix != row).astype(_f32)
    pm_ref[0] = jnp.clip(w, 1e-8, None) * mask
    idx_ref[0] = ix


def _run_topk(phi_h, psi_h):
    ph = phi_h.reshape(B, S, D)
    sh = psi_h.reshape(B, S, D)
    return pl.pallas_call(
        _topk_kernel,
        grid=(B, NI),
        in_specs=[
            pl.BlockSpec((1, TI, D), lambda b, i: (b, i, 0)),
            pl.BlockSpec((1, S, D), lambda b, i: (b, 0, 0)),
        ],
        out_specs=[
            pl.BlockSpec((1, TI, K), lambda b, i: (b, i, 0)),
            pl.BlockSpec((1, TI, K), lambda b, i: (b, i, 0)),
        ],
        out_shape=[
            jax.ShapeDtypeStruct((B, S, K), jnp.int32),
            jax.ShapeDtypeStruct((B, S, K), _f32),
        ],
    )(ph, sh)


def _gat_proj_kernel(x_ref, w_ref, as_ref, ad_ref, h_ref, ab_ref):
    hflat = _dot(x_ref[...], w_ref[...], 1, 1)
    h3 = hflat.reshape(TR, HEADS, D)
    a = jnp.sum(h3 * as_ref[...][None], axis=2)
    b = jnp.sum(h3 * ad_ref[...][None], axis=2)
    for hh in range(HEADS):
        h_ref[hh] = h3[:, hh, :]
    ab_ref[...] = jnp.concatenate([a, b], axis=1)


def _run_gat_proj(x, W, att_s, att_d):
    return pl.pallas_call(
        _gat_proj_kernel,
        grid=(N // TR,),
        in_specs=[
            pl.BlockSpec((TR, D), lambda i: (i, 0)),
            pl.BlockSpec((HEADS * D, D), lambda i: (0, 0)),
            pl.BlockSpec((HEADS, D), lambda i: (0, 0)),
            pl.BlockSpec((HEADS, D), lambda i: (0, 0)),
        ],
        out_specs=[
            pl.BlockSpec((HEADS, TR, D), lambda i: (0, i, 0)),
            pl.BlockSpec((TR, 2 * HEADS), lambda i: (i, 0)),
        ],
        out_shape=[
            jax.ShapeDtypeStruct((HEADS, N, D), _f32),
            jax.ShapeDtypeStruct((N, 2 * HEADS), _f32),
        ],
    )(x, W, att_s, att_d)


_bf16 = jnp.bfloat16


def _edgew_kernel(ab_ref, abf_ref, idx_ref, pm_ref, w_ref):
    a = ab_ref[...][:, :HEADS]
    bfull = abf_ref[...][:, HEADS:]
    idx = idx_ref[0]
    pm = pm_ref[0]
    iota = jax.lax.broadcasted_iota(jnp.int32, (TI, S), 1)
    cols = []
    for k in range(K):
        mk = (iota == idx[:, k][:, None]).astype(_f32)
        bg = _dot(mk, bfull, 1, 0)
        cols.append(pm[:, k][:, None] * jnp.exp(_leaky(a + bg)))
    w_ref[0] = jnp.concatenate(cols, axis=1)


def _run_edgew(ab, idx, pm):
    return pl.pallas_call(
        _edgew_kernel,
        grid=(B, NI),
        in_specs=[
            pl.BlockSpec((TI, 2 * HEADS), lambda b, i: (b * NI + i, 0)),
            pl.BlockSpec((S, 2 * HEADS), lambda b, i: (b, 0)),
            pl.BlockSpec((1, TI, K), lambda b, i: (b, i, 0)),
            pl.BlockSpec((1, TI, K), lambda b, i: (b, i, 0)),
        ],
        out_specs=pl.BlockSpec((1, TI, K * HEADS), lambda b, i: (b, i, 0)),
        out_shape=jax.ShapeDtypeStruct((B, S, K * HEADS), _f32),
    )(ab, ab, idx, pm)


def _agg_kernel(h_ref, w_ref, idx_ref, out_ref):
    hid = pl.program_id(1)

    htile = h_ref[0]
    wt = w_ref[0]
    idx16 = idx_ref[0].astype(jnp.int16)

    iota = jax.lax.broadcasted_iota(jnp.int16, (S, S), 1)
    cid = jax.lax.broadcasted_iota(jnp.int32, (1, K * HEADS), 1)
    E = jnp.zeros((S, S), _bf16)
    for k in range(K):
        match = iota == idx16[:, k][:, None]
        sel = (cid == k * HEADS + hid).astype(_f32)
        wk = jnp.sum(wt * sel, axis=1, keepdims=True)
        E = jnp.where(match, wk.astype(_bf16), E)

    x = jnp.concatenate(
        [htile.astype(_bf16), jnp.ones((S, 128), _bf16)], axis=1)
    out_ref[0, 0] = _dot(E, x, 0, 0)


def _run_agg(h, w, idx):
    return pl.pallas_call(
        _agg_kernel,
        grid=(B, HEADS),
        in_specs=[
            pl.BlockSpec((1, S, D), lambda b, h: (h, b, 0)),
            pl.BlockSpec((1, S, K * HEADS), lambda b, h: (b, 0, 0)),
            pl.BlockSpec((1, S, K), lambda b, h: (b, 0, 0)),
        ],
        out_specs=pl.BlockSpec((1, 1, S, D + 128), lambda b, h: (b, h, 0, 0)),
        out_shape=jax.ShapeDtypeStruct((B, HEADS, S, D + 128), _f32),
    )(h, w, idx)


TS = 512


def _norm_kernel(out_ref, x_ref):
    o = out_ref[0]
    at = o[:, :, D:D + 1]
    xn = jnp.mean(o[:, :, :D] / (at + 1e-8), axis=0)
    x_ref[0] = jnp.maximum(xn, 0.0)


def _run_norm(out):
    return pl.pallas_call(
        _norm_kernel,
        grid=(B, S // TS),
        in_specs=[
            pl.BlockSpec((1, HEADS, TS, D + 128), lambda b, j: (b, 0, j, 0)),
        ],
        out_specs=pl.BlockSpec((1, TS, D), lambda b, j: (b, j, 0)),
        out_shape=jax.ShapeDtypeStruct((B, S, D), _f32),
    )(out)


def _head_kernel(x_ref, pw_ref, w1_ref, b1_ref, w2_ref, b2_ref, o_ref):
    pooled = []
    for b in range(B):
        xb = x_ref[b]
        p = _dot(xb, pw_ref[...], 1, 1)
        p = p - jnp.max(p, axis=0, keepdims=True)
        al = jnp.exp(p)
        al = al / jnp.sum(al, axis=0, keepdims=True)
        pooled.append(_dot(al, xb, 0, 0))
    pooled = jnp.concatenate(pooled, axis=0)
    hmid = jnp.maximum(_dot(pooled, w1_ref[...], 1, 1) + b1_ref[...], 0.0)
    o_ref[...] = _dot(hmid, w2_ref[...], 1, 1) + b2_ref[...]


def _run_head(x, pool_W, W1, b1, W2, b2):
    return pl.pallas_call(
        _head_kernel,
        grid=(1,),
        in_specs=[
            pl.BlockSpec((B, S, D), lambda i: (0, 0, 0)),
            pl.BlockSpec((1, D), lambda i: (0, 0)),
            pl.BlockSpec((D // 2, D), lambda i: (0, 0)),
            pl.BlockSpec((1, D // 2), lambda i: (0, 0)),
            pl.BlockSpec((SEM, D // 2), lambda i: (0, 0)),
            pl.BlockSpec((1, SEM), lambda i: (0, 0)),
        ],
        out_specs=pl.BlockSpec((B, SEM), lambda i: (0, 0)),
        out_shape=jax.ShapeDtypeStruct((B, SEM), _f32),
    )(x, pool_W, W1, b1.reshape(1, -1), W2, b2.reshape(1, -1))


@jax.jit
def kernel(hidden_states, phi_W, psi_W, gat1_W, gat1_att, gat2_W, gat2_att,
           pool_W, proj_W1, proj_b1, proj_W2, proj_b2):
    x = hidden_states.reshape(N, D)
    phi_h, psi_h = _run_proj(x, phi_W, psi_W)
    idx, pm = _run_topk(phi_h, psi_h)

    for (W, att) in ((gat1_W, gat1_att), (gat2_W, gat2_att)):
        att_s = att[0, :, :D]
        att_d = att[0, :, D:]
        h, ab = _run_gat_proj(x, W, att_s, att_d)
        w = _run_edgew(ab, idx, pm)
        out = _run_agg(h, w, idx)
        xb = _run_norm(out)
        x = xb.reshape(N, D)

    return _run_head(xb, pool_W, proj_W1, proj_b1, proj_W2, proj_b2)


# speedup vs baseline: 1.3636x; 1.0251x over previous
import functools

import jax
import jax.numpy as jnp
from jax.experimental import pallas as pl
from jax.experimental.pallas import tpu as pltpu

B, S, D = 2, 2048, 768
HEADS = 4
K = 8
SEM = 512
N = B * S

TI = 512
NI = S // TI
TR = 1024

_f32 = jnp.float32


def _dot(a, b, ca, cb):
    return jax.lax.dot_general(
        a, b, (((ca,), (cb,)), ((), ())), preferred_element_type=_f32)


def _leaky(x):
    return jnp.where(x >= 0, x, 0.2 * x)


def _proj_kernel(x_ref, pw_ref, sw_ref, ph_ref, sh_ref):
    x = x_ref[...]
    ph_ref[...] = _dot(x, pw_ref[...], 1, 1)
    sh_ref[...] = _dot(x, sw_ref[...], 1, 1)


def _run_proj(x, phi_W, psi_W):
    return pl.pallas_call(
        _proj_kernel,
        grid=(N // TR,),
        in_specs=[
            pl.BlockSpec((TR, D), lambda i: (i, 0)),
            pl.BlockSpec((D, D), lambda i: (0, 0)),
            pl.BlockSpec((D, D), lambda i: (0, 0)),
        ],
        out_specs=[
            pl.BlockSpec((TR, D), lambda i: (i, 0)),
            pl.BlockSpec((TR, D), lambda i: (i, 0)),
        ],
        out_shape=[
            jax.ShapeDtypeStruct((N, D), _f32),
            jax.ShapeDtypeStruct((N, D), _f32),
        ],
    )(x, phi_W, psi_W)


def _topk_kernel(ph_ref, sh_ref, idx_ref, pm_ref):
    i = pl.program_id(1)
    phi = ph_ref[0]
    psi = sh_ref[0]
    aff = jnp.exp(_dot(phi, psi, 1, 1))
    iota = jax.lax.broadcasted_iota(jnp.int32, (TI, S), 1)
    work = aff
    vals = []
    idxs = []
    for _ in range(K):
        m = jnp.max(work, axis=1, keepdims=True)
        amin = jnp.min(jnp.where(work == m, iota, S), axis=1,
                       keepdims=True)
        vals.append(m)
        idxs.append(amin)
        work = jnp.where(iota == amin, -1.0, work)
    v = jnp.concatenate(vals, axis=1)
    ix = jnp.concatenate(idxs, axis=1)
    w = v / (jnp.sum(v, axis=1, keepdims=True) + 1e-8)
    row = i * TI + jax.lax.broadcasted_iota(jnp.int32, (TI, K), 0)
    mask = (ix != row).astype(_f32)
    pm_ref[0] = jnp.clip(w, 1e-8, None) * mask
    idx_ref[0] = ix


def _run_topk(phi_h, psi_h):
    ph = phi_h.reshape(B, S, D)
    sh = psi_h.reshape(B, S, D)
    return pl.pallas_call(
        _topk_kernel,
        grid=(B, NI),
        in_specs=[
            pl.BlockSpec((1, TI, D), lambda b, i: (b, i, 0)),
            pl.BlockSpec((1, S, D), lambda b, i: (b, 0, 0)),
        ],
        out_specs=[
            pl.BlockSpec((1, TI, K), lambda b, i: (b, i, 0)),
            pl.BlockSpec((1, TI, K), lambda b, i: (b, i, 0)),
        ],
        out_shape=[
            jax.ShapeDtypeStruct((B, S, K), jnp.int32),
            jax.ShapeDtypeStruct((B, S, K), _f32),
        ],
    )(ph, sh)


def _gat_proj_kernel(x_ref, w_ref, as_ref, ad_ref, h_ref, ab_ref):
    hflat = _dot(x_ref[...], w_ref[...], 1, 1)
    h3 = hflat.reshape(TR, HEADS, D)
    a = jnp.sum(h3 * as_ref[...][None], axis=2)
    b = jnp.sum(h3 * ad_ref[...][None], axis=2)
    for hh in range(HEADS):
        h_ref[hh] = h3[:, hh, :]
    ab_ref[...] = jnp.concatenate([a, b], axis=1)


def _run_gat_proj(x, W, att_s, att_d):
    return pl.pallas_call(
        _gat_proj_kernel,
        grid=(N // TR,),
        in_specs=[
            pl.BlockSpec((TR, D), lambda i: (i, 0)),
            pl.BlockSpec((HEADS * D, D), lambda i: (0, 0)),
            pl.BlockSpec((HEADS, D), lambda i: (0, 0)),
            pl.BlockSpec((HEADS, D), lambda i: (0, 0)),
        ],
        out_specs=[
            pl.BlockSpec((HEADS, TR, D), lambda i: (0, i, 0)),
            pl.BlockSpec((TR, 2 * HEADS), lambda i: (i, 0)),
        ],
        out_shape=[
            jax.ShapeDtypeStruct((HEADS, N, D), _f32),
            jax.ShapeDtypeStruct((N, 2 * HEADS), _f32),
        ],
    )(x, W, att_s, att_d)


_bf16 = jnp.bfloat16


def _edgew_kernel(ab_ref, abf_ref, idx_ref, pm_ref, w_ref):
    a = ab_ref[...][:, :HEADS]
    bfull = abf_ref[...][:, HEADS:]
    idx = idx_ref[0]
    pm = pm_ref[0]
    iota = jax.lax.broadcasted_iota(jnp.int32, (TI, S), 1)
    cols = []
    for k in range(K):
        mk = (iota == idx[:, k][:, None]).astype(_f32)
        bg = _dot(mk, bfull, 1, 0)
        cols.append(pm[:, k][:, None] * jnp.exp(_leaky(a + bg)))
    w_ref[0] = jnp.concatenate(cols, axis=1)


def _run_edgew(ab, idx, pm):
    return pl.pallas_call(
        _edgew_kernel,
        grid=(B, NI),
        in_specs=[
            pl.BlockSpec((TI, 2 * HEADS), lambda b, i: (b * NI + i, 0)),
            pl.BlockSpec((S, 2 * HEADS), lambda b, i: (b, 0)),
            pl.BlockSpec((1, TI, K), lambda b, i: (b, i, 0)),
            pl.BlockSpec((1, TI, K), lambda b, i: (b, i, 0)),
        ],
        out_specs=pl.BlockSpec((1, TI, K * HEADS), lambda b, i: (b, i, 0)),
        out_shape=jax.ShapeDtypeStruct((B, S, K * HEADS), _f32),
    )(ab, ab, idx, pm)


def _agg_kernel(h_ref, w_ref, idx_ref, out_ref):
    hid = pl.program_id(1)

    htile = h_ref[0]
    wt = w_ref[0]
    idx16 = idx_ref[0].astype(jnp.int16)

    iota = jax.lax.broadcasted_iota(jnp.int16, (S, S), 1)
    cid = jax.lax.broadcasted_iota(jnp.int32, (1, K * HEADS), 1)
    E = jnp.zeros((S, S), _bf16)
    for k in range(K):
        match = iota == idx16[:, k][:, None]
        sel = (cid == k * HEADS + hid).astype(_f32)
        wk = jnp.sum(wt * sel, axis=1, keepdims=True)
        E = jnp.where(match, wk.astype(_bf16), E)

    x = jnp.concatenate(
        [htile.astype(_bf16), jnp.ones((S, 128), _bf16)], axis=1)
    out_ref[0, 0] = _dot(E, x, 0, 0)


def _run_agg(h, w, idx):
    return pl.pallas_call(
        _agg_kernel,
        grid=(B, HEADS),
        in_specs=[
            pl.BlockSpec((1, S, D), lambda b, h: (h, b, 0)),
            pl.BlockSpec((1, S, K * HEADS), lambda b, h: (b, 0, 0)),
            pl.BlockSpec((1, S, K), lambda b, h: (b, 0, 0)),
        ],
        out_specs=pl.BlockSpec((1, 1, S, D + 128), lambda b, h: (b, h, 0, 0)),
        out_shape=jax.ShapeDtypeStruct((B, HEADS, S, D + 128), _f32),
    )(h, w, idx)


TS = 512


def _norm_kernel(out_ref, x_ref):
    o = out_ref[0]
    at = o[:, :, D:D + 1]
    xn = jnp.mean(o[:, :, :D] / (at + 1e-8), axis=0)
    x_ref[0] = jnp.maximum(xn, 0.0)


def _run_norm(out):
    return pl.pallas_call(
        _norm_kernel,
        grid=(B, S // TS),
        in_specs=[
            pl.BlockSpec((1, HEADS, TS, D + 128), lambda b, j: (b, 0, j, 0)),
        ],
        out_specs=pl.BlockSpec((1, TS, D), lambda b, j: (b, j, 0)),
        out_shape=jax.ShapeDtypeStruct((B, S, D), _f32),
    )(out)


def _head_kernel(x_ref, pw_ref, w1_ref, b1_ref, w2_ref, b2_ref, o_ref):
    pooled = []
    for b in range(B):
        xb = x_ref[b]
        p = _dot(xb, pw_ref[...], 1, 1)
        p = p - jnp.max(p, axis=0, keepdims=True)
        al = jnp.exp(p)
        al = al / jnp.sum(al, axis=0, keepdims=True)
        pooled.append(_dot(al, xb, 0, 0))
    pooled = jnp.concatenate(pooled, axis=0)
    hmid = jnp.maximum(_dot(pooled, w1_ref[...], 1, 1) + b1_ref[...], 0.0)
    o_ref[...] = _dot(hmid, w2_ref[...], 1, 1) + b2_ref[...]


def _run_head(x, pool_W, W1, b1, W2, b2):
    return pl.pallas_call(
        _head_kernel,
        grid=(1,),
        in_specs=[
            pl.BlockSpec((B, S, D), lambda i: (0, 0, 0)),
            pl.BlockSpec((1, D), lambda i: (0, 0)),
            pl.BlockSpec((D // 2, D), lambda i: (0, 0)),
            pl.BlockSpec((1, D // 2), lambda i: (0, 0)),
            pl.BlockSpec((SEM, D // 2), lambda i: (0, 0)),
            pl.BlockSpec((1, SEM), lambda i: (0, 0)),
        ],
        out_specs=pl.BlockSpec((B, SEM), lambda i: (0, 0)),
        out_shape=jax.ShapeDtypeStruct((B, SEM), _f32),
    )(x, pool_W, W1, b1.reshape(1, -1), W2, b2.reshape(1, -1))


@jax.jit
def kernel(hidden_states, phi_W, psi_W, gat1_W, gat1_att, gat2_W, gat2_att,
           pool_W, proj_W1, proj_b1, proj_W2, proj_b2):
    x = hidden_states.reshape(N, D)
    phi_h, psi_h = _run_proj(x, phi_W, psi_W)
    idx, pm = _run_topk(phi_h, psi_h)

    for (W, att) in ((gat1_W, gat1_att), (gat2_W, gat2_att)):
        att_s = att[0, :, :D]
        att_d = att[0, :, D:]
        h, ab = _run_gat_proj(x, W, att_s, att_d)
        w = _run_edgew(ab, idx, pm)
        out = _run_agg(h, w, idx)
        xb = _run_norm(out)
        x = xb.reshape(N, D)

    return _run_head(xb, pool_W, proj_W1, proj_b1, proj_W2, proj_b2)
